# baseline (speedup 1.0000x reference)
import jax
import jax.numpy as jnp
from jax.experimental import pallas as pl
from jax.experimental.pallas import tpu as pltpu

_ROWS = 512


def _add_body(x_ref, pe_ref, out_ref):
    out_ref[...] = x_ref[...] + pe_ref[...][None]


def kernel(x, pos_emb):
    B, T, C = x.shape
    grid = (T // _ROWS,)
    return pl.pallas_call(
        _add_body,
        grid=grid,
        in_specs=[
            pl.BlockSpec((B, _ROWS, C), lambda t: (0, t, 0)),
            pl.BlockSpec((_ROWS, C), lambda t: (t, 0)),
        ],
        out_specs=pl.BlockSpec((B, _ROWS, C), lambda t: (0, t, 0)),
        out_shape=jax.ShapeDtypeStruct((B, T, C), x.dtype),
    )(x, pos_emb)
